# baseline (speedup 1.0000x reference)
import jax
import jax.numpy as jnp
from jax.experimental import pallas as pl
from jax.experimental.pallas import tpu as pltpu

_PERIODS = 12
_RESOLUTION_S = 3600.0
_SLOTS = _PERIODS + 1
_NSEM = 8


def _body(s_ref, past, latest, upd, dmax, dzero, sems):
    shift = s_ref[0]

    @pl.when(shift > 0)
    def _():
        cps = []
        for k in range(_PERIODS):
            cps.append(pltpu.make_async_copy(past.at[k + 1], upd.at[k], sems.at[k % _NSEM]))
        cps.append(pltpu.make_async_copy(latest, upd.at[_PERIODS], sems.at[_PERIODS % _NSEM]))
        cps.append(pltpu.make_async_copy(past.at[1], dmax, sems.at[(_PERIODS + 1) % _NSEM]))
        cps.append(pltpu.make_async_copy(latest, dzero, sems.at[(_PERIODS + 2) % _NSEM]))
        for c in cps:
            c.start()
        for c in cps:
            c.wait()

    @pl.when(shift == 0)
    def _():
        cps = []
        for k in range(_SLOTS):
            cps.append(pltpu.make_async_copy(past.at[k], upd.at[k], sems.at[k % _NSEM]))
        cps.append(pltpu.make_async_copy(past.at[0], dmax, sems.at[(_SLOTS) % _NSEM]))
        cps.append(pltpu.make_async_copy(past.at[_PERIODS], dzero, sems.at[(_SLOTS + 1) % _NSEM]))
        for c in cps:
            c.start()
        for c in cps:
            c.wait()


def kernel(past, latest, dt_mod_freq, timedelta_seconds):
    dt = dt_mod_freq[0] + jnp.float32(timedelta_seconds)
    is_update_step = dt >= _RESOLUTION_S
    new_dt = jnp.where(is_update_step, dt - _RESOLUTION_S, dt)
    shift = is_update_step.astype(jnp.int32).reshape((1,))
    field = jax.ShapeDtypeStruct(latest.shape, latest.dtype)
    updated_past, diag_max, diag_zero = pl.pallas_call(
        _body,
        in_specs=[
            pl.BlockSpec(memory_space=pltpu.MemorySpace.SMEM),
            pl.BlockSpec(memory_space=pl.ANY),
            pl.BlockSpec(memory_space=pl.ANY),
        ],
        out_specs=[
            pl.BlockSpec(memory_space=pl.ANY),
            pl.BlockSpec(memory_space=pl.ANY),
            pl.BlockSpec(memory_space=pl.ANY),
        ],
        out_shape=[jax.ShapeDtypeStruct(past.shape, past.dtype), field, field],
        scratch_shapes=[pltpu.SemaphoreType.DMA((_NSEM,))],
    )(shift, past, latest)
    return updated_past, diag_max, diag_zero, new_dt
